# baseline (speedup 1.0000x reference)
import functools

import jax
import jax.numpy as jnp
from jax import lax
from jax.experimental import pallas as pl
from jax.experimental.pallas import tpu as pltpu
from jax.experimental.pallas import tpu_sc as plsc

V = 100000
D = 128
B = 4096
MOM = 0.5
NC, NS, L = 2, 16, 16
NW = NC * NS
R = V // NW
RCAP = 3136
TRASHR = RCAP - 1
CH = 128
CAP = B + 2 * L
TRASH = CAP - 1
DB = D // L

_mesh = plsc.VectorSubcoreMesh(core_axis_name="c", subcore_axis_name="s")


def _blend_normalize(mrow, xrow, r):
    acc = jnp.zeros((L,), jnp.float32)
    vs = []
    for dblk in range(DB):
        s = pl.ds(dblk * L, L)
        v = mrow[r, s] * MOM + xrow[r, s] * (1.0 - MOM)
        vs.append(v)
        acc = acc + v * v
    ss = jnp.full((L,), jnp.sum(acc), jnp.float32)
    bits = plsc.bitcast(ss, jnp.int32)
    guess = plsc.bitcast(
        jnp.full((L,), 0x5F3759DF, jnp.int32) - (bits >> 1), jnp.float32)
    for _ in range(3):
        guess = guess * (1.5 - 0.5 * ss * guess * guess)
    for dblk in range(DB):
        mrow[r, pl.ds(dblk * L, L)] = vs[dblk] * guess


@functools.partial(
    pl.kernel,
    out_type=(),
    mesh=_mesh,
    compiler_params=pltpu.CompilerParams(needs_layout_passes=False),
    scratch_types=[
        pltpu.VMEM((B,), jnp.int32),
        pltpu.VMEM((RCAP,), jnp.int32),
        pltpu.VMEM((CAP,), jnp.int32),
        pltpu.VMEM((CAP,), jnp.int32),
        pltpu.VMEM((B // CH, CH), jnp.int32),
        pltpu.VMEM((CH, D), jnp.float32),
        pltpu.VMEM((CH, D), jnp.float32),
        pltpu.SemaphoreType.DMA,
        pltpu.SemaphoreType.DMA,
    ],
)
def _sc_update(x_hbm, y_hbm, mem_hbm, out_ref,
               y_v, win_v, pos_v, idx_v, idx2, mrow, xrow, semA, semB):
    wid = lax.axis_index("s") * NC + lax.axis_index("c")
    lo = wid * R
    hi = lo + R
    lanes = lax.iota(jnp.int32, L)
    onev = jnp.full((L,), 1, jnp.int32)
    zerov = jnp.full((L,), 0, jnp.int32)
    lov = jnp.full((L,), lo, jnp.int32)
    hiv = jnp.full((L,), hi, jnp.int32)
    sinkr = jnp.full((L,), TRASHR, jnp.int32)

    pltpu.sync_copy(y_hbm, y_v)

    @pl.loop(0, RCAP // L, unroll=8)
    def clear(b):
        win_v[pl.ds(b * L, L)] = jnp.full((L,), -1, jnp.int32)

    @pl.loop(0, B // L, unroll=4)
    def winners(i):
        yv = y_v[pl.ds(i * L, L)]
        m = (yv >= lov) & (yv < hiv)
        local = jnp.where(m, yv - lov, sinkr)
        pos = i * L + lanes
        plsc.store_scatter(win_v, [local], pos)
        g = plsc.load_gather(win_v, [local])
        bad0 = m & (g < pos)
        nb0 = plsc.all_reduce_population_count(bad0)[0]

        def cond(carry):
            return carry[0] > 0

        def body(carry):
            _, bad = carry
            slots = jnp.where(bad, local, sinkr)
            plsc.store_scatter(win_v, [slots], pos)
            g2 = plsc.load_gather(win_v, [slots])
            bad2 = bad & (g2 < pos)
            return (plsc.all_reduce_population_count(bad2)[0], bad2)

        lax.while_loop(cond, body, (nb0, bad0))

    @pl.loop(0, B // L, init_carry=jnp.int32(0), unroll=4)
    def compact(i, cnt):
        yv = y_v[pl.ds(i * L, L)]
        m = (yv >= lov) & (yv < hiv)
        local = jnp.where(m, yv - lov, sinkr)
        pos = i * L + lanes
        g = plsc.load_gather(win_v, [local])
        win = m & (g == pos)
        mi = jnp.where(win, onev, zerov)
        slots = jnp.where(win, plsc.cumsum(mi) + jnp.full((L,), cnt - 1, jnp.int32),
                          jnp.full((L,), TRASH, jnp.int32))
        plsc.store_scatter(pos_v, [slots], pos)
        plsc.store_scatter(idx_v, [slots], yv)
        return cnt + plsc.all_reduce_population_count(win)[0]

    cnt = compact

    @pl.when(cnt > 0)
    def _():
        nfull = cnt // CH
        tail = cnt - nfull * CH
        ntg = (tail + L - 1) // L
        pend = nfull * CH + ntg * L

        last_idx = idx_v[pl.ds(cnt - 1, L)][0]
        last_pos = pos_v[pl.ds(cnt - 1, L)][0]

        @pl.loop(cnt // L, pend // L)
        def fill(b):
            base = b * L
            live = base + lanes < jnp.full((L,), cnt, jnp.int32)
            cur_i = idx_v[pl.ds(base, L)]
            cur_p = pos_v[pl.ds(base, L)]
            idx_v[pl.ds(base, L)] = jnp.where(live, cur_i, jnp.full((L,), last_idx, jnp.int32))
            pos_v[pl.ds(base, L)] = jnp.where(live, cur_p, jnp.full((L,), last_pos, jnp.int32))

        @pl.loop(0, nfull)
        def chunk(c):
            off = c * CH
            for b in range(CH // L):
                idx2[c, pl.ds(b * L, L)] = idx_v[pl.ds(off + b * L, L)]
            gm = pltpu.async_copy(mem_hbm.at[idx2.at[c]], mrow, semA)
            gx = pltpu.async_copy(x_hbm.at[pos_v.at[pl.ds(off, CH)]], xrow, semB)
            gm.wait()
            gx.wait()

            @pl.loop(0, CH)
            def row(r):
                _blend_normalize(mrow, xrow, r)

            pltpu.async_copy(mrow, out_ref.at[idx2.at[c]], semA).wait()

        tbase = nfull * CH

        @pl.loop(0, ntg)
        def tg_fire(g):
            iv = idx_v[pl.ds(tbase + g * L, L)]
            pv = pos_v[pl.ds(tbase + g * L, L)]
            pltpu.make_async_copy(mem_hbm.at[iv], mrow.at[pl.ds(g * L, L)],
                                  semA).start()
            pltpu.make_async_copy(x_hbm.at[pv], xrow.at[pl.ds(g * L, L)],
                                  semB).start()

        @pl.loop(0, ntg)
        def tg_drain(g):
            iv = idx_v[pl.ds(tbase + g * L, L)]
            pv = pos_v[pl.ds(tbase + g * L, L)]
            pltpu.make_async_copy(mem_hbm.at[iv], mrow.at[pl.ds(g * L, L)],
                                  semA).wait()
            pltpu.make_async_copy(x_hbm.at[pv], xrow.at[pl.ds(g * L, L)],
                                  semB).wait()

        @pl.loop(0, ntg * L)
        def trow(r):
            _blend_normalize(mrow, xrow, r)

        @pl.loop(0, ntg)
        def tg_scatter(g):
            iv = idx_v[pl.ds(tbase + g * L, L)]
            pltpu.make_async_copy(mrow.at[pl.ds(g * L, L)], out_ref.at[iv],
                                  semA).start()

        @pl.loop(0, ntg)
        def tg_sdrain(g):
            iv = idx_v[pl.ds(tbase + g * L, L)]
            pltpu.make_async_copy(mrow.at[pl.ds(g * L, L)], out_ref.at[iv],
                                  semA).wait()


_CPB = 1000


def _copy_body(src, dst):
    dst[...] = src[...]


_tc_copy = pl.pallas_call(
    _copy_body,
    out_shape=jax.ShapeDtypeStruct((V, D), jnp.float32),
    grid=(V // _CPB,),
    in_specs=[pl.BlockSpec((_CPB, D), lambda i: (i, 0))],
    out_specs=pl.BlockSpec((_CPB, D), lambda i: (i, 0)),
)


def kernel(x, x2, y, memory):
    mem_ref = jax.new_ref(_tc_copy(memory))
    _sc_update(x, y, memory, mem_ref)
    return (x, x2, mem_ref[...])


# speedup vs baseline: 1.5092x; 1.5092x over previous
import functools

import jax
import jax.numpy as jnp
from jax import lax
from jax.experimental import pallas as pl
from jax.experimental.pallas import tpu as pltpu
from jax.experimental.pallas import tpu_sc as plsc

V = 100000
D = 128
B = 4096
MOM = 0.5
NC, NS, L = 2, 16, 16
NW = NC * NS
R = V // NW
RCAP = 3136
TRASHR = RCAP - 1
CH = 128
CAP = B + 2 * L
TRASH = CAP - 1
DB = D // L

_mesh = plsc.VectorSubcoreMesh(core_axis_name="c", subcore_axis_name="s")


def _blend_normalize(mrow, xrow, r):
    acc = jnp.zeros((L,), jnp.float32)
    vs = []
    for dblk in range(DB):
        s = pl.ds(dblk * L, L)
        v = mrow[r, s] * MOM + xrow[r, s] * (1.0 - MOM)
        vs.append(v)
        acc = acc + v * v
    ss = jnp.full((L,), jnp.sum(acc), jnp.float32)
    bits = plsc.bitcast(ss, jnp.int32)
    guess = plsc.bitcast(
        jnp.full((L,), 0x5F3759DF, jnp.int32) - (bits >> 1), jnp.float32)
    for _ in range(3):
        guess = guess * (1.5 - 0.5 * ss * guess * guess)
    for dblk in range(DB):
        mrow[r, pl.ds(dblk * L, L)] = vs[dblk] * guess


@functools.partial(
    pl.kernel,
    out_type=(),
    mesh=_mesh,
    compiler_params=pltpu.CompilerParams(needs_layout_passes=False),
    scratch_types=[
        pltpu.VMEM((B,), jnp.int32),
        pltpu.VMEM((RCAP,), jnp.int32),
        pltpu.VMEM((CAP,), jnp.int32),
        pltpu.VMEM((CAP,), jnp.int32),
        pltpu.VMEM((B // CH, CH), jnp.int32),
        pltpu.VMEM((CH, D), jnp.float32),
        pltpu.VMEM((CH, D), jnp.float32),
        pltpu.SemaphoreType.DMA,
        pltpu.SemaphoreType.DMA,
    ],
)
def _sc_update(x_hbm, y_hbm, mem_hbm, out_ref,
               y_v, win_v, pos_v, idx_v, idx2, mrow, xrow, semA, semB):
    wid = lax.axis_index("s") * NC + lax.axis_index("c")
    lo = wid * R
    hi = lo + R
    lanes = lax.iota(jnp.int32, L)
    onev = jnp.full((L,), 1, jnp.int32)
    zerov = jnp.full((L,), 0, jnp.int32)
    lov = jnp.full((L,), lo, jnp.int32)
    hiv = jnp.full((L,), hi, jnp.int32)
    sinkr = jnp.full((L,), TRASHR, jnp.int32)

    pltpu.sync_copy(y_hbm, y_v)

    @pl.loop(0, RCAP // L, unroll=8)
    def clear(b):
        win_v[pl.ds(b * L, L)] = jnp.full((L,), -1, jnp.int32)

    @pl.loop(0, B // L, unroll=4)
    def winners(i):
        yv = y_v[pl.ds(i * L, L)]
        m = (yv >= lov) & (yv < hiv)
        local = jnp.where(m, yv - lov, sinkr)
        pos = i * L + lanes
        plsc.store_scatter(win_v, [local], pos)
        g = plsc.load_gather(win_v, [local])
        bad0 = m & (g < pos)
        nb0 = plsc.all_reduce_population_count(bad0)[0]

        def cond(carry):
            return carry[0] > 0

        def body(carry):
            _, bad = carry
            slots = jnp.where(bad, local, sinkr)
            plsc.store_scatter(win_v, [slots], pos)
            g2 = plsc.load_gather(win_v, [slots])
            bad2 = bad & (g2 < pos)
            return (plsc.all_reduce_population_count(bad2)[0], bad2)

        lax.while_loop(cond, body, (nb0, bad0))

    @pl.loop(0, B // L, init_carry=jnp.int32(0), unroll=4)
    def compact(i, cnt):
        yv = y_v[pl.ds(i * L, L)]
        m = (yv >= lov) & (yv < hiv)
        local = jnp.where(m, yv - lov, sinkr)
        pos = i * L + lanes
        g = plsc.load_gather(win_v, [local])
        win = m & (g == pos)
        mi = jnp.where(win, onev, zerov)
        slots = jnp.where(win, plsc.cumsum(mi) + jnp.full((L,), cnt - 1, jnp.int32),
                          jnp.full((L,), TRASH, jnp.int32))
        plsc.store_scatter(pos_v, [slots], pos)
        plsc.store_scatter(idx_v, [slots], yv)
        return cnt + plsc.all_reduce_population_count(win)[0]

    cnt = compact

    @pl.when(cnt > 0)
    def _():
        nfull = cnt // CH
        tail = cnt - nfull * CH
        ntg = (tail + L - 1) // L
        pend = nfull * CH + ntg * L

        last_idx = idx_v[pl.ds(cnt - 1, L)][0]
        last_pos = pos_v[pl.ds(cnt - 1, L)][0]

        @pl.loop(cnt // L, pend // L)
        def fill(b):
            base = b * L
            live = base + lanes < jnp.full((L,), cnt, jnp.int32)
            cur_i = idx_v[pl.ds(base, L)]
            cur_p = pos_v[pl.ds(base, L)]
            idx_v[pl.ds(base, L)] = jnp.where(live, cur_i, jnp.full((L,), last_idx, jnp.int32))
            pos_v[pl.ds(base, L)] = jnp.where(live, cur_p, jnp.full((L,), last_pos, jnp.int32))

        @pl.loop(0, nfull)
        def chunk(c):
            off = c * CH
            for b in range(CH // L):
                idx2[c, pl.ds(b * L, L)] = idx_v[pl.ds(off + b * L, L)]
            gm = pltpu.async_copy(mem_hbm.at[idx2.at[c]], mrow, semA)
            gx = pltpu.async_copy(x_hbm.at[pos_v.at[pl.ds(off, CH)]], xrow, semB)
            gm.wait()
            gx.wait()

            @pl.loop(0, CH)
            def row(r):
                _blend_normalize(mrow, xrow, r)

            pltpu.async_copy(mrow, out_ref.at[idx2.at[c]], semA).wait()

        tbase = nfull * CH

        @pl.loop(0, ntg)
        def tg_fire(g):
            iv = idx_v[pl.ds(tbase + g * L, L)]
            pv = pos_v[pl.ds(tbase + g * L, L)]
            pltpu.make_async_copy(mem_hbm.at[iv], mrow.at[pl.ds(g * L, L)],
                                  semA).start()
            pltpu.make_async_copy(x_hbm.at[pv], xrow.at[pl.ds(g * L, L)],
                                  semB).start()

        @pl.loop(0, ntg)
        def tg_drain(g):
            iv = idx_v[pl.ds(tbase + g * L, L)]
            pv = pos_v[pl.ds(tbase + g * L, L)]
            pltpu.make_async_copy(mem_hbm.at[iv], mrow.at[pl.ds(g * L, L)],
                                  semA).wait()
            pltpu.make_async_copy(x_hbm.at[pv], xrow.at[pl.ds(g * L, L)],
                                  semB).wait()

        @pl.loop(0, ntg * L)
        def trow(r):
            _blend_normalize(mrow, xrow, r)

        @pl.loop(0, ntg)
        def tg_scatter(g):
            iv = idx_v[pl.ds(tbase + g * L, L)]
            pltpu.make_async_copy(mrow.at[pl.ds(g * L, L)], out_ref.at[iv],
                                  semA).start()

        @pl.loop(0, ntg)
        def tg_sdrain(g):
            iv = idx_v[pl.ds(tbase + g * L, L)]
            pltpu.make_async_copy(mrow.at[pl.ds(g * L, L)], out_ref.at[iv],
                                  semA).wait()


def kernel(x, x2, y, memory):
    mem_ref = jax.new_ref(memory)
    _sc_update(x, y, memory, mem_ref)
    return (x, x2, mem_ref[...])


# speedup vs baseline: 1.6776x; 1.1116x over previous
import functools

import jax
import jax.numpy as jnp
from jax import lax
from jax.experimental import pallas as pl
from jax.experimental.pallas import tpu as pltpu
from jax.experimental.pallas import tpu_sc as plsc

V = 100000
D = 128
B = 4096
MOM = 0.5
NC, NS, L = 2, 16, 16
NW = NC * NS
R = V // NW
RCAP = 3136
TRASHR = RCAP - 1
CH = 128
CAP = B + 2 * L
TRASH = CAP - 1
DB = D // L

_mesh = plsc.VectorSubcoreMesh(core_axis_name="c", subcore_axis_name="s")


def _blend_normalize(mrow, xrow, r):
    acc = jnp.zeros((L,), jnp.float32)
    vs = []
    for dblk in range(DB):
        s = pl.ds(dblk * L, L)
        v = mrow[r, s] * MOM + xrow[r, s] * (1.0 - MOM)
        vs.append(v)
        acc = acc + v * v
    ss = jnp.full((L,), jnp.sum(acc), jnp.float32)
    bits = plsc.bitcast(ss, jnp.int32)
    guess = plsc.bitcast(
        jnp.full((L,), 0x5F3759DF, jnp.int32) - (bits >> 1), jnp.float32)
    for _ in range(3):
        guess = guess * (1.5 - 0.5 * ss * guess * guess)
    for dblk in range(DB):
        mrow[r, pl.ds(dblk * L, L)] = vs[dblk] * guess


@functools.partial(
    pl.kernel,
    out_type=(),
    mesh=_mesh,
    compiler_params=pltpu.CompilerParams(needs_layout_passes=False),
    scratch_types=[
        pltpu.VMEM((B,), jnp.int32),
        pltpu.VMEM((RCAP,), jnp.int32),
        pltpu.VMEM((CAP,), jnp.int32),
        pltpu.VMEM((CAP,), jnp.int32),
        pltpu.VMEM((B // CH, CH), jnp.int32),
        pltpu.VMEM((CH, D), jnp.float32),
        pltpu.VMEM((CH, D), jnp.float32),
        pltpu.SemaphoreType.DMA,
        pltpu.SemaphoreType.DMA,
    ],
)
def _sc_update(x_hbm, y_hbm, mem_hbm, out_ref,
               y_v, win_v, pos_v, idx_v, idx2, mrow, xrow, semA, semB):
    wid = lax.axis_index("s") * NC + lax.axis_index("c")
    lo = wid * R
    hi = lo + R
    lanes = lax.iota(jnp.int32, L)
    onev = jnp.full((L,), 1, jnp.int32)
    zerov = jnp.full((L,), 0, jnp.int32)
    lov = jnp.full((L,), lo, jnp.int32)
    hiv = jnp.full((L,), hi, jnp.int32)
    sinkr = jnp.full((L,), TRASHR, jnp.int32)

    pltpu.sync_copy(y_hbm, y_v)

    @pl.loop(0, B // L, init_carry=jnp.int32(0), unroll=4)
    def compact(i, cnt):
        yv = y_v[pl.ds(i * L, L)]
        m = (yv >= lov) & (yv < hiv)
        mi = jnp.where(m, onev, zerov)
        slots = jnp.where(m, plsc.cumsum(mi) + jnp.full((L,), cnt - 1, jnp.int32),
                          jnp.full((L,), TRASH, jnp.int32))
        plsc.store_scatter(pos_v, [slots], i * L + lanes)
        plsc.store_scatter(idx_v, [slots], yv)
        return cnt + plsc.all_reduce_population_count(m)[0]

    cnt0 = compact

    nblk = (cnt0 + L - 1) // L

    @pl.loop(0, nblk)
    def w0(b):
        base = b * L
        live = base + lanes < jnp.full((L,), cnt0, jnp.int32)
        local = jnp.where(live, idx_v[pl.ds(base, L)] - lov, sinkr)
        plsc.store_scatter(win_v, [local], jnp.full((L,), -1, jnp.int32))

    @pl.loop(0, nblk)
    def w1(b):
        base = b * L
        live = base + lanes < jnp.full((L,), cnt0, jnp.int32)
        local = jnp.where(live, idx_v[pl.ds(base, L)] - lov, sinkr)
        pos = pos_v[pl.ds(base, L)]
        plsc.store_scatter(win_v, [local], pos)
        g = plsc.load_gather(win_v, [local])
        bad0 = live & (g < pos)
        nb0 = plsc.all_reduce_population_count(bad0)[0]

        def cond(carry):
            return carry[0] > 0

        def body(carry):
            _, bad = carry
            slots = jnp.where(bad, local, sinkr)
            plsc.store_scatter(win_v, [slots], pos)
            g2 = plsc.load_gather(win_v, [slots])
            bad2 = bad & (g2 < pos)
            return (plsc.all_reduce_population_count(bad2)[0], bad2)

        lax.while_loop(cond, body, (nb0, bad0))

    @pl.loop(0, nblk, init_carry=jnp.int32(0))
    def w2(b, cnt_w):
        base = b * L
        live = base + lanes < jnp.full((L,), cnt0, jnp.int32)
        yv2 = idx_v[pl.ds(base, L)]
        local = jnp.where(live, yv2 - lov, sinkr)
        pos = pos_v[pl.ds(base, L)]
        g = plsc.load_gather(win_v, [local])
        win = live & (g == pos)
        mi = jnp.where(win, onev, zerov)
        slots = jnp.where(win, plsc.cumsum(mi) + jnp.full((L,), cnt_w - 1, jnp.int32),
                          jnp.full((L,), TRASH, jnp.int32))
        plsc.store_scatter(pos_v, [slots], pos)
        plsc.store_scatter(idx_v, [slots], yv2)
        return cnt_w + plsc.all_reduce_population_count(win)[0]

    cnt = w2

    @pl.when(cnt > 0)
    def _():
        nfull = cnt // CH
        tail = cnt - nfull * CH
        ntg = (tail + L - 1) // L
        pend = nfull * CH + ntg * L

        last_idx = idx_v[pl.ds(cnt - 1, L)][0]
        last_pos = pos_v[pl.ds(cnt - 1, L)][0]

        @pl.loop(cnt // L, pend // L)
        def fill(b):
            base = b * L
            live = base + lanes < jnp.full((L,), cnt, jnp.int32)
            cur_i = idx_v[pl.ds(base, L)]
            cur_p = pos_v[pl.ds(base, L)]
            idx_v[pl.ds(base, L)] = jnp.where(live, cur_i, jnp.full((L,), last_idx, jnp.int32))
            pos_v[pl.ds(base, L)] = jnp.where(live, cur_p, jnp.full((L,), last_pos, jnp.int32))

        @pl.loop(0, nfull)
        def chunk(c):
            off = c * CH
            for b in range(CH // L):
                idx2[c, pl.ds(b * L, L)] = idx_v[pl.ds(off + b * L, L)]
            gm = pltpu.async_copy(mem_hbm.at[idx2.at[c]], mrow, semA)
            gx = pltpu.async_copy(x_hbm.at[pos_v.at[pl.ds(off, CH)]], xrow, semB)
            gm.wait()
            gx.wait()

            @pl.loop(0, CH)
            def row(r):
                _blend_normalize(mrow, xrow, r)

            pltpu.async_copy(mrow, out_ref.at[idx2.at[c]], semA).wait()

        tbase = nfull * CH

        @pl.loop(0, ntg)
        def tg_fire(g):
            iv = idx_v[pl.ds(tbase + g * L, L)]
            pv = pos_v[pl.ds(tbase + g * L, L)]
            pltpu.make_async_copy(mem_hbm.at[iv], mrow.at[pl.ds(g * L, L)],
                                  semA).start()
            pltpu.make_async_copy(x_hbm.at[pv], xrow.at[pl.ds(g * L, L)],
                                  semB).start()

        @pl.loop(0, ntg)
        def tg_drain(g):
            iv = idx_v[pl.ds(tbase + g * L, L)]
            pv = pos_v[pl.ds(tbase + g * L, L)]
            pltpu.make_async_copy(mem_hbm.at[iv], mrow.at[pl.ds(g * L, L)],
                                  semA).wait()
            pltpu.make_async_copy(x_hbm.at[pv], xrow.at[pl.ds(g * L, L)],
                                  semB).wait()

        @pl.loop(0, ntg * L)
        def trow(r):
            _blend_normalize(mrow, xrow, r)

        @pl.loop(0, ntg)
        def tg_scatter(g):
            iv = idx_v[pl.ds(tbase + g * L, L)]
            pltpu.make_async_copy(mrow.at[pl.ds(g * L, L)], out_ref.at[iv],
                                  semA).start()

        @pl.loop(0, ntg)
        def tg_sdrain(g):
            iv = idx_v[pl.ds(tbase + g * L, L)]
            pltpu.make_async_copy(mrow.at[pl.ds(g * L, L)], out_ref.at[iv],
                                  semA).wait()


def kernel(x, x2, y, memory):
    mem_ref = jax.new_ref(memory)
    _sc_update(x, y, memory, mem_ref)
    return (x, x2, mem_ref[...])


# speedup vs baseline: 1.6956x; 1.0108x over previous
import functools

import jax
import jax.numpy as jnp
from jax import lax
from jax.experimental import pallas as pl
from jax.experimental.pallas import tpu as pltpu
from jax.experimental.pallas import tpu_sc as plsc

V = 100000
D = 128
B = 4096
MOM = 0.5
NC, NS, L = 2, 16, 16
NW = NC * NS
R = V // NW
RCAP = 3136
TRASHR = RCAP - 1
CH = 128
CAP = B + 2 * L
TRASH = CAP - 1
DB = D // L

_mesh = plsc.VectorSubcoreMesh(core_axis_name="c", subcore_axis_name="s")


def _blend_normalize(mrow, xrow, r):
    acc = jnp.zeros((L,), jnp.float32)
    vs = []
    for dblk in range(DB):
        s = pl.ds(dblk * L, L)
        v = mrow[r, s] * MOM + xrow[r, s] * (1.0 - MOM)
        vs.append(v)
        acc = acc + v * v
    ss = jnp.full((L,), jnp.sum(acc), jnp.float32)
    bits = plsc.bitcast(ss, jnp.int32)
    guess = plsc.bitcast(
        jnp.full((L,), 0x5F3759DF, jnp.int32) - (bits >> 1), jnp.float32)
    for _ in range(2):
        guess = guess * (1.5 - 0.5 * ss * guess * guess)
    for dblk in range(DB):
        mrow[r, pl.ds(dblk * L, L)] = vs[dblk] * guess


@functools.partial(
    pl.kernel,
    out_type=(),
    mesh=_mesh,
    compiler_params=pltpu.CompilerParams(needs_layout_passes=False),
    scratch_types=[
        pltpu.VMEM((B,), jnp.int32),
        pltpu.VMEM((RCAP,), jnp.int32),
        pltpu.VMEM((CAP,), jnp.int32),
        pltpu.VMEM((CAP,), jnp.int32),
        pltpu.VMEM((B // CH, CH), jnp.int32),
        pltpu.VMEM((CH, D), jnp.float32),
        pltpu.VMEM((CH, D), jnp.float32),
        pltpu.SemaphoreType.DMA,
        pltpu.SemaphoreType.DMA,
    ],
)
def _sc_update(x_hbm, y_hbm, mem_hbm, out_ref,
               y_v, win_v, pos_v, idx_v, idx2, mrow, xrow, semA, semB):
    wid = lax.axis_index("s") * NC + lax.axis_index("c")
    lo = wid * R
    hi = lo + R
    lanes = lax.iota(jnp.int32, L)
    onev = jnp.full((L,), 1, jnp.int32)
    zerov = jnp.full((L,), 0, jnp.int32)
    lov = jnp.full((L,), lo, jnp.int32)
    hiv = jnp.full((L,), hi, jnp.int32)
    sinkr = jnp.full((L,), TRASHR, jnp.int32)

    pltpu.sync_copy(y_hbm, y_v)

    @pl.loop(0, B // L, init_carry=jnp.int32(0), unroll=8)
    def compact(i, cnt):
        yv = y_v[pl.ds(i * L, L)]
        m = (yv >= lov) & (yv < hiv)
        mi = jnp.where(m, onev, zerov)
        slots = jnp.where(m, plsc.cumsum(mi) + jnp.full((L,), cnt - 1, jnp.int32),
                          jnp.full((L,), TRASH, jnp.int32))
        plsc.store_scatter(pos_v, [slots], i * L + lanes)
        plsc.store_scatter(idx_v, [slots], yv)
        return cnt + plsc.all_reduce_population_count(m)[0]

    cnt0 = compact

    nblk = (cnt0 + L - 1) // L

    @pl.loop(0, nblk)
    def w0(b):
        base = b * L
        live = base + lanes < jnp.full((L,), cnt0, jnp.int32)
        local = jnp.where(live, idx_v[pl.ds(base, L)] - lov, sinkr)
        plsc.store_scatter(win_v, [local], jnp.full((L,), -1, jnp.int32))

    @pl.loop(0, nblk)
    def w1(b):
        base = b * L
        live = base + lanes < jnp.full((L,), cnt0, jnp.int32)
        local = jnp.where(live, idx_v[pl.ds(base, L)] - lov, sinkr)
        pos = pos_v[pl.ds(base, L)]
        plsc.store_scatter(win_v, [local], pos)
        g = plsc.load_gather(win_v, [local])
        bad0 = live & (g < pos)
        nb0 = plsc.all_reduce_population_count(bad0)[0]

        def cond(carry):
            return carry[0] > 0

        def body(carry):
            _, bad = carry
            slots = jnp.where(bad, local, sinkr)
            plsc.store_scatter(win_v, [slots], pos)
            g2 = plsc.load_gather(win_v, [slots])
            bad2 = bad & (g2 < pos)
            return (plsc.all_reduce_population_count(bad2)[0], bad2)

        lax.while_loop(cond, body, (nb0, bad0))

    @pl.loop(0, nblk, init_carry=jnp.int32(0))
    def w2(b, cnt_w):
        base = b * L
        live = base + lanes < jnp.full((L,), cnt0, jnp.int32)
        yv2 = idx_v[pl.ds(base, L)]
        local = jnp.where(live, yv2 - lov, sinkr)
        pos = pos_v[pl.ds(base, L)]
        g = plsc.load_gather(win_v, [local])
        win = live & (g == pos)
        mi = jnp.where(win, onev, zerov)
        slots = jnp.where(win, plsc.cumsum(mi) + jnp.full((L,), cnt_w - 1, jnp.int32),
                          jnp.full((L,), TRASH, jnp.int32))
        plsc.store_scatter(pos_v, [slots], pos)
        plsc.store_scatter(idx_v, [slots], yv2)
        return cnt_w + plsc.all_reduce_population_count(win)[0]

    cnt = w2

    @pl.when(cnt > 0)
    def _():
        nfull = cnt // CH
        tail = cnt - nfull * CH
        ntg = (tail + L - 1) // L
        pend = nfull * CH + ntg * L

        last_idx = idx_v[pl.ds(cnt - 1, L)][0]
        last_pos = pos_v[pl.ds(cnt - 1, L)][0]

        @pl.loop(cnt // L, pend // L)
        def fill(b):
            base = b * L
            live = base + lanes < jnp.full((L,), cnt, jnp.int32)
            cur_i = idx_v[pl.ds(base, L)]
            cur_p = pos_v[pl.ds(base, L)]
            idx_v[pl.ds(base, L)] = jnp.where(live, cur_i, jnp.full((L,), last_idx, jnp.int32))
            pos_v[pl.ds(base, L)] = jnp.where(live, cur_p, jnp.full((L,), last_pos, jnp.int32))

        @pl.loop(0, nfull)
        def chunk(c):
            off = c * CH
            for b in range(CH // L):
                idx2[c, pl.ds(b * L, L)] = idx_v[pl.ds(off + b * L, L)]
            gm = pltpu.async_copy(mem_hbm.at[idx2.at[c]], mrow, semA)
            gx = pltpu.async_copy(x_hbm.at[pos_v.at[pl.ds(off, CH)]], xrow, semB)
            gm.wait()
            gx.wait()

            @pl.loop(0, CH)
            def row(r):
                _blend_normalize(mrow, xrow, r)

            pltpu.async_copy(mrow, out_ref.at[idx2.at[c]], semA).wait()

        tbase = nfull * CH

        @pl.loop(0, ntg)
        def tg_fire(g):
            iv = idx_v[pl.ds(tbase + g * L, L)]
            pv = pos_v[pl.ds(tbase + g * L, L)]
            pltpu.make_async_copy(mem_hbm.at[iv], mrow.at[pl.ds(g * L, L)],
                                  semA).start()
            pltpu.make_async_copy(x_hbm.at[pv], xrow.at[pl.ds(g * L, L)],
                                  semB).start()

        @pl.loop(0, ntg)
        def tg_drain(g):
            iv = idx_v[pl.ds(tbase + g * L, L)]
            pv = pos_v[pl.ds(tbase + g * L, L)]
            pltpu.make_async_copy(mem_hbm.at[iv], mrow.at[pl.ds(g * L, L)],
                                  semA).wait()
            pltpu.make_async_copy(x_hbm.at[pv], xrow.at[pl.ds(g * L, L)],
                                  semB).wait()

        @pl.loop(0, ntg * L)
        def trow(r):
            _blend_normalize(mrow, xrow, r)

        @pl.loop(0, ntg)
        def tg_scatter(g):
            iv = idx_v[pl.ds(tbase + g * L, L)]
            pltpu.make_async_copy(mrow.at[pl.ds(g * L, L)], out_ref.at[iv],
                                  semA).start()

        @pl.loop(0, ntg)
        def tg_sdrain(g):
            iv = idx_v[pl.ds(tbase + g * L, L)]
            pltpu.make_async_copy(mrow.at[pl.ds(g * L, L)], out_ref.at[iv],
                                  semA).wait()


def kernel(x, x2, y, memory):
    mem_ref = jax.new_ref(memory)
    _sc_update(x, y, memory, mem_ref)
    return (x, x2, mem_ref[...])
